# baseline (speedup 1.0000x reference)
import functools

import numpy as np

import jax
import jax.numpy as jnp
from jax import lax
from jax.experimental import pallas as pl
from jax.experimental.pallas import tpu as pltpu
from jax.experimental.pallas import tpu_sc as plsc

N = 10000
E = 320000
D = 128
DE = 16

_NSUB = 16
_EPS_PER = E // _NSUB
_K = 80
_NCH = _EPS_PER // _K
_PADN = 10240
_NCHP = 256
_ZROWS = _PADN // _NSUB
_ZB = 128

_PERM = np.concatenate(
    [np.arange(16) + 32 * q for q in range(D // 32)]
    + [np.arange(16) + 32 * q + 16 for q in range(D // 32)]
)


def _eproj(ea_even, ea_odd, w_edge, b_edge):
    bm = 640
    grid = (ea_even.shape[0] // bm,)

    def pack(r):
        bits = lax.bitcast_convert_type(r, jnp.int32)
        rnd = lax.shift_right_arithmetic(
            bits + 0x7FFF + jnp.bitwise_and(
                lax.shift_right_arithmetic(bits, 16), 1), 16)
        lo = jnp.bitwise_and(rnd[:, : D // 2], 0xFFFF)
        hi = lax.shift_left(rnd[:, D // 2:], 16)
        return jnp.bitwise_or(lo, hi)

    def body(ee_ref, eo_ref, w_ref, b_ref, o_ref):
        re = (jnp.dot(ee_ref[...], w_ref[...],
                      preferred_element_type=jnp.float32) + b_ref[...])
        ro = (jnp.dot(eo_ref[...], w_ref[...],
                      preferred_element_type=jnp.float32) + b_ref[...])
        o_ref[...] = jnp.concatenate([pack(re), pack(ro)], axis=1)

    return pl.pallas_call(
        body,
        grid=grid,
        in_specs=[
            pl.BlockSpec((bm, DE), lambda i: (i, 0)),
            pl.BlockSpec((bm, DE), lambda i: (i, 0)),
            pl.BlockSpec((DE, D), lambda i: (0, 0)),
            pl.BlockSpec((1, D), lambda i: (0, 0)),
        ],
        out_specs=pl.BlockSpec((bm, D), lambda i: (i, 0)),
        out_shape=jax.ShapeDtypeStruct((ea_even.shape[0], D), jnp.int32),
    )(ea_even, ea_odd, w_edge, b_edge.reshape(1, D))


def _out_transform(x_stack, agg_stack, w_nn, b_nn, eps, relu):
    bm = 1000
    grid = (x_stack.shape[0] // bm,)

    def body(x_ref, a_ref, w_ref, b_ref, e_ref, o_ref):
        z = (1.0 + e_ref[0, 0]) * x_ref[...] + a_ref[...]
        r = jnp.dot(z, w_ref[...], preferred_element_type=jnp.float32) + b_ref[...]
        o_ref[...] = jnp.maximum(r, 0.0) if relu else r

    return pl.pallas_call(
        body,
        grid=grid,
        in_specs=[
            pl.BlockSpec((bm, D), lambda i: (i, 0)),
            pl.BlockSpec((bm, D), lambda i: (i, 0)),
            pl.BlockSpec((D, D), lambda i: (0, 0)),
            pl.BlockSpec((1, D), lambda i: (0, 0)),
            pl.BlockSpec((1, 1), lambda i: (0, 0)),
        ],
        out_specs=pl.BlockSpec((bm, D), lambda i: (i, 0)),
        out_shape=jax.ShapeDtypeStruct((x_stack.shape[0], D), jnp.float32),
    )(x_stack, agg_stack, w_nn, b_nn.reshape(1, D), eps.reshape(1, 1))


def _sc_aggregate(x_all, src_all, dst_all, ep_all):
    mesh = plsc.VectorSubcoreMesh(core_axis_name="c", subcore_axis_name="s")

    @functools.partial(
        pl.kernel,
        out_type=jax.ShapeDtypeStruct((2, _PADN, D), jnp.float32),
        mesh=mesh,
        scratch_types=[
            pltpu.VMEM((2, _K), jnp.int32),
            pltpu.VMEM((2, _K), jnp.int32),
            pltpu.VMEM((2, _K, D), jnp.float32),
            pltpu.VMEM((2, _K // 2, D), jnp.int32),
            pltpu.VMEM_SHARED((_PADN, D), jnp.float32),
            pltpu.SemaphoreType.DMA,
            pltpu.SemaphoreType.DMA,
            pltpu.SemaphoreType.DMA,
            pltpu.SemaphoreType.DMA,
            pltpu.SemaphoreType.DMA,
            pltpu.SemaphoreType.DMA,
            pltpu.SemaphoreType.DMA,
            pltpu.SemaphoreType.DMA,
        ],
    )
    def k(x_hbm, src_hbm, dst_hbm, ep_hbm, out_hbm,
          srcv, dstv, gx_v, ep_v, acc_sh,
          sem_src, sem_dst, sem_g0, sem_g1, sem_e0, sem_e1, sem_s0, sem_s1):
        cid = lax.axis_index("c")
        sid = lax.axis_index("s")
        sem_g = (sem_g0, sem_g1)
        sem_e = (sem_e0, sem_e1)
        sem_s = (sem_s0, sem_s1)

        ebase = (cid * _NSUB + sid) * _EPS_PER
        pbase = (cid * _NSUB + sid) * (_EPS_PER // 2)

        zvec = jnp.zeros((16,), jnp.float32)

        def zrow(r, carry):
            for j in range(D // 16):
                gx_v[1, r, pl.ds(j * 16, 16)] = zvec
            return carry

        lax.fori_loop(0, _K, zrow, 0)

        def zacc(t, carry):
            pltpu.sync_copy(gx_v.at[1],
                            acc_sh.at[pl.ds(sid * _ZROWS + t * _K, _K)])
            return carry

        lax.fori_loop(0, _ZROWS // _K, zacc, 0)

        def issue_src(ch, sl):
            pltpu.async_copy(src_hbm.at[pl.ds(ebase + ch * _K, _K)],
                             srcv.at[sl], sem_src)

        def wait_src(ch, sl):
            pltpu.make_async_copy(src_hbm.at[pl.ds(ebase + ch * _K, _K)],
                                  srcv.at[sl], sem_src).wait()

        def issue_dst(ch, sl):
            pltpu.async_copy(dst_hbm.at[pl.ds(ebase + ch * _K, _K)],
                             dstv.at[sl], sem_dst)

        def wait_dst(ch, sl):
            pltpu.make_async_copy(dst_hbm.at[pl.ds(ebase + ch * _K, _K)],
                                  dstv.at[sl], sem_dst).wait()

        def issue_in(ch, sl):
            pltpu.async_copy(x_hbm.at[srcv.at[sl]], gx_v.at[sl], sem_g[sl])
            pltpu.async_copy(ep_hbm.at[pl.ds(pbase + ch * (_K // 2), _K // 2)],
                             ep_v.at[sl], sem_e[sl])

        def wait_in(ch, sl):
            pltpu.make_async_copy(x_hbm.at[srcv.at[sl]], gx_v.at[sl],
                                  sem_g[sl]).wait()
            pltpu.make_async_copy(
                ep_hbm.at[pl.ds(pbase + ch * (_K // 2), _K // 2)],
                ep_v.at[sl], sem_e[sl]).wait()

        zero16 = jnp.zeros((16,), jnp.float32)
        himask = jnp.full((16,), -65536, jnp.int32)

        def compute(sl):
            def crow(r, c2):
                for p in range(2):
                    for q in range(D // 32):
                        ew = ep_v[sl, r, pl.ds(64 * p + 16 * q, 16)]
                        elo = lax.bitcast_convert_type(
                            jnp.left_shift(ew, 16), jnp.float32)
                        ehi = lax.bitcast_convert_type(
                            jnp.bitwise_and(ew, himask), jnp.float32)
                        slo = pl.ds(32 * q, 16)
                        shi = pl.ds(32 * q + 16, 16)
                        gx_v[sl, 2 * r + p, slo] = jnp.maximum(
                            gx_v[sl, 2 * r + p, slo] + elo, zero16)
                        gx_v[sl, 2 * r + p, shi] = jnp.maximum(
                            gx_v[sl, 2 * r + p, shi] + ehi, zero16)
                return c2

            lax.fori_loop(0, _K // 2, crow, 0)

        def scatter(sl):
            pltpu.async_copy(gx_v.at[sl], acc_sh.at[dstv.at[sl]],
                             sem_s[sl], add=True)

        def wait_scatter(sl):
            pltpu.make_async_copy(gx_v.at[sl], acc_sh.at[dstv.at[sl]],
                                  sem_s[sl]).wait()

        issue_src(0, 0)
        wait_src(0, 0)
        issue_in(0, 0)
        issue_src(1, 1)
        pltpu.sync_copy(dst_hbm.at[pl.ds(ebase, _K)], dstv.at[0])
        pltpu.async_copy(gx_v.at[1], acc_sh.at[dstv.at[0]], sem_s1,
                         add=True)
        issue_dst(0, 0)
        plsc.subcore_barrier()

        def phase(ph, sl, not_g, not_s):
            nsl = 1 - sl
            wait_scatter(nsl)

            @pl.when(not_g)
            def _():
                wait_src(ph + 1, nsl)
                issue_in(ph + 1, nsl)

            wait_in(ph, sl)

            @pl.when(not_s)
            def _():
                issue_src(ph + 2, sl)

            @pl.when(not_g)
            def _():
                issue_dst(ph + 1, nsl)

            compute(sl)
            wait_dst(ph, sl)
            scatter(sl)

        t_last = _NCH // 2 - 1

        def step(t, carry):
            a_not_s = t < t_last
            b_not = t < t_last
            phase(2 * t, 0, jnp.bool_(True), a_not_s)
            phase(2 * t + 1, 1, b_not, b_not)
            return carry

        lax.fori_loop(0, _NCH // 2, step, 0)
        wait_scatter(1)
        plsc.subcore_barrier()

        def dump(t, carry):
            r0 = sid * _ZROWS + t * _ZB
            pltpu.sync_copy(acc_sh.at[pl.ds(r0, _ZB)],
                            out_hbm.at[cid, pl.ds(r0, _ZB)])
            return carry

        lax.fori_loop(0, _ZROWS // _ZB, dump, 0)

    return k(x_all, src_all, dst_all, ep_all)


def kernel(x_author, x_paper, edge_index_a2p, edge_index_p2a,
           edge_attr_a2p, edge_attr_p2a, W_edge, b_edge, W_nn, b_nn, eps):
    src_all = jnp.concatenate([
        edge_index_a2p[0].astype(jnp.int32),
        edge_index_p2a[0].astype(jnp.int32) + N,
    ])
    dst_all = jnp.concatenate([
        edge_index_a2p[1].astype(jnp.int32),
        edge_index_p2a[1].astype(jnp.int32),
    ])
    ea_all = jnp.concatenate([edge_attr_a2p, edge_attr_p2a], axis=0)
    ep_all = _eproj(ea_all[0::2], ea_all[1::2],
                    W_edge[:, _PERM], b_edge[_PERM])

    xa, xp = x_author, x_paper
    for layer in range(2):
        x_all = jnp.concatenate([xa, xp], axis=0)
        agg = _sc_aggregate(x_all, src_all, dst_all, ep_all)[:, :N, :]
        x_stack = jnp.concatenate([xp, xa], axis=0)
        new_stack = _out_transform(x_stack, agg.reshape(2 * N, D),
                                   W_nn, b_nn, eps, relu=(layer == 0))
        xp, xa = new_stack[:N], new_stack[N:]
    return (xa, xp)


# speedup vs baseline: 2.3359x; 2.3359x over previous
import functools

import jax
import jax.numpy as jnp
from jax import lax
from jax.experimental import pallas as pl
from jax.experimental.pallas import tpu as pltpu
from jax.experimental.pallas import tpu_sc as plsc

N = 10000
E = 320000
D = 128
DE = 16

_NSUB = 16
_EPS_PER = E // _NSUB
_K = 80
_NCH = _EPS_PER // _K
_PADN = 10240
_NCHP = 256
_ZROWS = _PADN // _NSUB
_ZB = 128


def _eproj(ea_all, w_edge, b_edge):
    bm = 1280
    grid = (ea_all.shape[0] // bm,)

    def body(ea_ref, w_ref, b_ref, o_ref):
        o_ref[...] = (
            jnp.dot(ea_ref[...], w_ref[...], preferred_element_type=jnp.float32)
            + b_ref[...]
        )

    return pl.pallas_call(
        body,
        grid=grid,
        in_specs=[
            pl.BlockSpec((bm, DE), lambda i: (i, 0)),
            pl.BlockSpec((DE, D), lambda i: (0, 0)),
            pl.BlockSpec((1, D), lambda i: (0, 0)),
        ],
        out_specs=pl.BlockSpec((bm, D), lambda i: (i, 0)),
        out_shape=jax.ShapeDtypeStruct((ea_all.shape[0], D), jnp.float32),
    )(ea_all, w_edge, b_edge.reshape(1, D))


def _out_transform(xp, xa, agg, w_nn, b_nn, eps, relu):
    bm = 1000
    nb = N // bm
    grid = (2 * nb,)

    def body(xp_ref, xa_ref, a_ref, w_ref, b_ref, e_ref, o_ref):
        x = jnp.where(pl.program_id(0) < nb, xp_ref[...], xa_ref[...])
        z = (1.0 + e_ref[0, 0]) * x + a_ref[...]
        r = jnp.dot(z, w_ref[...], preferred_element_type=jnp.float32) + b_ref[...]
        o_ref[...] = jnp.maximum(r, 0.0) if relu else r

    return pl.pallas_call(
        body,
        grid=grid,
        in_specs=[
            pl.BlockSpec((bm, D), lambda i: (jnp.minimum(i, nb - 1), 0)),
            pl.BlockSpec((bm, D), lambda i: (jnp.maximum(i - nb, 0), 0)),
            pl.BlockSpec((bm, D), lambda i: (i, 0)),
            pl.BlockSpec((D, D), lambda i: (0, 0)),
            pl.BlockSpec((1, D), lambda i: (0, 0)),
            pl.BlockSpec((1, 1), lambda i: (0, 0)),
        ],
        out_specs=pl.BlockSpec((bm, D), lambda i: (i, 0)),
        out_shape=jax.ShapeDtypeStruct((2 * N, D), jnp.float32),
    )(xp, xa, agg.reshape(2 * N, D), w_nn, b_nn.reshape(1, D),
      eps.reshape(1, 1))


def _sc_aggregate(x_all, src_all, dst_all, ep_all):
    mesh = plsc.VectorSubcoreMesh(core_axis_name="c", subcore_axis_name="s")

    @functools.partial(
        pl.kernel,
        out_type=jax.ShapeDtypeStruct((2, N, D), jnp.float32),
        mesh=mesh,
        scratch_types=[
            pltpu.VMEM((2, _K), jnp.int32),
            pltpu.VMEM((2, _K), jnp.int32),
            pltpu.VMEM((2, _K, D), jnp.float32),
            pltpu.VMEM((2, _K, D), jnp.float32),
            pltpu.VMEM_SHARED((N, D), jnp.float32),
            pltpu.SemaphoreType.DMA,
            pltpu.SemaphoreType.DMA,
            pltpu.SemaphoreType.DMA,
            pltpu.SemaphoreType.DMA,
            pltpu.SemaphoreType.DMA,
            pltpu.SemaphoreType.DMA,
            pltpu.SemaphoreType.DMA,
            pltpu.SemaphoreType.DMA,
        ],
    )
    def k(x_hbm, src_hbm, dst_hbm, ep_hbm, out_hbm,
          srcv, dstv, gx_v, ep_v, acc_sh,
          sem_src, sem_dst, sem_g0, sem_g1, sem_e0, sem_e1, sem_s0, sem_s1):
        cid = lax.axis_index("c")
        sid = lax.axis_index("s")
        sem_g = (sem_g0, sem_g1)
        sem_e = (sem_e0, sem_e1)
        sem_s = (sem_s0, sem_s1)

        ebase = (cid * _NSUB + sid) * _EPS_PER

        zvec = jnp.zeros((16,), jnp.float32)

        def zrow(r, carry):
            for j in range(D // 16):
                gx_v[1, r, pl.ds(j * 16, 16)] = zvec
            return carry

        lax.fori_loop(0, _K, zrow, 0)

        nz = jnp.where(sid < (N // _K) % _NSUB, (N // _K) // _NSUB + 1,
                       (N // _K) // _NSUB)

        def zacc(t, carry):
            pltpu.sync_copy(gx_v.at[1],
                            acc_sh.at[pl.ds((sid + t * _NSUB) * _K, _K)])
            return carry

        lax.fori_loop(0, nz, zacc, 0)

        def issue_src(ch, sl):
            pltpu.async_copy(src_hbm.at[pl.ds(ebase + ch * _K, _K)],
                             srcv.at[sl], sem_src)

        def wait_src(ch, sl):
            pltpu.make_async_copy(src_hbm.at[pl.ds(ebase + ch * _K, _K)],
                                  srcv.at[sl], sem_src).wait()

        def issue_dst(ch, sl):
            pltpu.async_copy(dst_hbm.at[pl.ds(ebase + ch * _K, _K)],
                             dstv.at[sl], sem_dst)

        def wait_dst(ch, sl):
            pltpu.make_async_copy(dst_hbm.at[pl.ds(ebase + ch * _K, _K)],
                                  dstv.at[sl], sem_dst).wait()

        def issue_in(ch, sl):
            pltpu.async_copy(x_hbm.at[srcv.at[sl]], gx_v.at[sl], sem_g[sl])
            pltpu.async_copy(ep_hbm.at[pl.ds(ebase + ch * _K, _K)],
                             ep_v.at[sl], sem_e[sl])

        def wait_in(ch, sl):
            pltpu.make_async_copy(x_hbm.at[srcv.at[sl]], gx_v.at[sl],
                                  sem_g[sl]).wait()
            pltpu.make_async_copy(ep_hbm.at[pl.ds(ebase + ch * _K, _K)],
                                  ep_v.at[sl], sem_e[sl]).wait()

        def compute(sl):
            def crow(e, c2):
                for j in range(D // 16):
                    s_ = pl.ds(j * 16, 16)
                    gx_v[sl, e, s_] = jnp.maximum(
                        gx_v[sl, e, s_] + ep_v[sl, e, s_], 0.0)
                return c2

            lax.fori_loop(0, _K, crow, 0)

        def scatter(sl):
            pltpu.async_copy(gx_v.at[sl], acc_sh.at[dstv.at[sl]],
                             sem_s[sl], add=True)

        def wait_scatter(sl):
            pltpu.make_async_copy(gx_v.at[sl], acc_sh.at[dstv.at[sl]],
                                  sem_s[sl]).wait()

        issue_src(0, 0)
        wait_src(0, 0)
        issue_in(0, 0)
        issue_src(1, 1)
        pltpu.sync_copy(dst_hbm.at[pl.ds(ebase, _K)], dstv.at[0])
        pltpu.async_copy(gx_v.at[1], acc_sh.at[dstv.at[0]], sem_s1,
                         add=True)
        issue_dst(0, 0)
        plsc.subcore_barrier()

        def phase(ph, sl, not_g, not_s):
            nsl = 1 - sl
            wait_scatter(nsl)

            @pl.when(not_g)
            def _():
                wait_src(ph + 1, nsl)
                issue_in(ph + 1, nsl)

            wait_in(ph, sl)

            @pl.when(not_s)
            def _():
                issue_src(ph + 2, sl)

            @pl.when(not_g)
            def _():
                issue_dst(ph + 1, nsl)

            compute(sl)
            wait_dst(ph, sl)
            scatter(sl)

        t_last = _NCH // 2 - 1

        def step(t, carry):
            a_not_s = t < t_last
            b_not = t < t_last
            phase(2 * t, 0, jnp.bool_(True), a_not_s)
            phase(2 * t + 1, 1, b_not, b_not)
            return carry

        lax.fori_loop(0, _NCH // 2, step, 0)
        wait_scatter(1)
        plsc.subcore_barrier()

        def dump(t, carry):
            r0 = (sid + t * _NSUB) * _K
            pltpu.sync_copy(acc_sh.at[pl.ds(r0, _K)],
                            out_hbm.at[cid, pl.ds(r0, _K)])
            return carry

        lax.fori_loop(0, nz, dump, 0)

    return k(x_all, src_all, dst_all, ep_all)


def kernel(x_author, x_paper, edge_index_a2p, edge_index_p2a,
           edge_attr_a2p, edge_attr_p2a, W_edge, b_edge, W_nn, b_nn, eps):
    src_all = jnp.concatenate([
        edge_index_a2p[0].astype(jnp.int32),
        edge_index_p2a[0].astype(jnp.int32) + N,
    ])
    dst_all = jnp.concatenate([
        edge_index_a2p[1].astype(jnp.int32),
        edge_index_p2a[1].astype(jnp.int32),
    ])
    ea_all = jnp.concatenate([edge_attr_a2p, edge_attr_p2a], axis=0)
    ep_all = _eproj(ea_all, W_edge, b_edge)

    xa, xp = x_author, x_paper
    for layer in range(2):
        x_all = jnp.concatenate([xa, xp], axis=0)
        agg = _sc_aggregate(x_all, src_all, dst_all, ep_all)
        new_stack = _out_transform(xp, xa, agg, W_nn, b_nn, eps,
                                   relu=(layer == 0))
        xp, xa = new_stack[:N], new_stack[N:]
    return (xa, xp)


# speedup vs baseline: 2.3364x; 1.0002x over previous
import functools

import jax
import jax.numpy as jnp
from jax import lax
from jax.experimental import pallas as pl
from jax.experimental.pallas import tpu as pltpu
from jax.experimental.pallas import tpu_sc as plsc

N = 10000
E = 320000
D = 128
DE = 16

_NSUB = 16
_EPS_PER = E // _NSUB
_K = 80
_NCH = _EPS_PER // _K
_NZCH = N // _K


def _eproj(ea_all, w_edge, b_edge):
    bm = 1280
    grid = (ea_all.shape[0] // bm,)

    def body(ea_ref, w_ref, b_ref, o_ref):
        o_ref[...] = (
            jnp.dot(ea_ref[...], w_ref[...], preferred_element_type=jnp.float32)
            + b_ref[...]
        )

    return pl.pallas_call(
        body,
        grid=grid,
        in_specs=[
            pl.BlockSpec((bm, DE), lambda i: (i, 0)),
            pl.BlockSpec((DE, D), lambda i: (0, 0)),
            pl.BlockSpec((1, D), lambda i: (0, 0)),
        ],
        out_specs=pl.BlockSpec((bm, D), lambda i: (i, 0)),
        out_shape=jax.ShapeDtypeStruct((ea_all.shape[0], D), jnp.float32),
    )(ea_all, w_edge, b_edge.reshape(1, D))


def _out_transform(xp, xa, agg, w_nn, b_nn, eps, relu):
    bm = 1000
    nb = N // bm
    grid = (2 * nb,)

    def body(xp_ref, xa_ref, a_ref, w_ref, b_ref, e_ref, o_ref):
        x = jnp.where(pl.program_id(0) < nb, xp_ref[...], xa_ref[...])
        z = (1.0 + e_ref[0, 0]) * x + a_ref[...]
        r = jnp.dot(z, w_ref[...], preferred_element_type=jnp.float32) + b_ref[...]
        o_ref[...] = jnp.maximum(r, 0.0) if relu else r

    return pl.pallas_call(
        body,
        grid=grid,
        in_specs=[
            pl.BlockSpec((bm, D), lambda i: (jnp.minimum(i, nb - 1), 0)),
            pl.BlockSpec((bm, D), lambda i: (jnp.maximum(i - nb, 0), 0)),
            pl.BlockSpec((bm, D), lambda i: (i, 0)),
            pl.BlockSpec((D, D), lambda i: (0, 0)),
            pl.BlockSpec((1, D), lambda i: (0, 0)),
            pl.BlockSpec((1, 1), lambda i: (0, 0)),
        ],
        out_specs=pl.BlockSpec((bm, D), lambda i: (i, 0)),
        out_shape=jax.ShapeDtypeStruct((2 * N, D), jnp.float32),
    )(xp, xa, agg.reshape(2 * N, D), w_nn, b_nn.reshape(1, D),
      eps.reshape(1, 1))


def _sc_aggregate(x_all, src_all, dst_all, ep_all):
    mesh = plsc.VectorSubcoreMesh(core_axis_name="c", subcore_axis_name="s")

    @functools.partial(
        pl.kernel,
        out_type=jax.ShapeDtypeStruct((2, N, D), jnp.float32),
        mesh=mesh,
        scratch_types=[
            pltpu.VMEM((2, _K), jnp.int32),
            pltpu.VMEM((2, _K), jnp.int32),
            pltpu.VMEM((2, _K, D), jnp.float32),
            pltpu.VMEM((2, _K, D), jnp.float32),
            pltpu.VMEM_SHARED((N, D), jnp.float32),
            pltpu.SemaphoreType.DMA,
            pltpu.SemaphoreType.DMA,
            pltpu.SemaphoreType.DMA,
            pltpu.SemaphoreType.DMA,
            pltpu.SemaphoreType.DMA,
            pltpu.SemaphoreType.DMA,
            pltpu.SemaphoreType.DMA,
            pltpu.SemaphoreType.DMA,
        ],
    )
    def k(x_hbm, src_hbm, dst_hbm, ep_hbm, out_hbm,
          srcv, dstv, gx_v, ep_v, acc_sh,
          sem_src, sem_dst, sem_g0, sem_g1, sem_e0, sem_e1, sem_s0, sem_s1):
        cid = lax.axis_index("c")
        sid = lax.axis_index("s")
        sem_g = (sem_g0, sem_g1)
        sem_e = (sem_e0, sem_e1)
        sem_s = (sem_s0, sem_s1)

        ebase = (cid * _NSUB + sid) * _EPS_PER

        zvec = jnp.zeros((16,), jnp.float32)

        def zrow(r, carry):
            for j in range(D // 16):
                gx_v[1, r, pl.ds(j * 16, 16)] = zvec
            return carry

        lax.fori_loop(0, _K, zrow, 0)

        nz = jnp.where(sid < _NZCH % _NSUB, _NZCH // _NSUB + 1,
                       _NZCH // _NSUB)

        def zacc(t, carry):
            pltpu.sync_copy(gx_v.at[1],
                            acc_sh.at[pl.ds((sid + t * _NSUB) * _K, _K)])
            return carry

        lax.fori_loop(0, nz, zacc, 0)

        def issue_src(ch, sl):
            pltpu.async_copy(src_hbm.at[pl.ds(ebase + ch * _K, _K)],
                             srcv.at[sl], sem_src)

        def wait_src(ch, sl):
            pltpu.make_async_copy(src_hbm.at[pl.ds(ebase + ch * _K, _K)],
                                  srcv.at[sl], sem_src).wait()

        def issue_dst(ch, sl):
            pltpu.async_copy(dst_hbm.at[pl.ds(ebase + ch * _K, _K)],
                             dstv.at[sl], sem_dst)

        def wait_dst(ch, sl):
            pltpu.make_async_copy(dst_hbm.at[pl.ds(ebase + ch * _K, _K)],
                                  dstv.at[sl], sem_dst).wait()

        def issue_in(ch, sl):
            pltpu.async_copy(x_hbm.at[srcv.at[sl]], gx_v.at[sl], sem_g[sl])
            pltpu.async_copy(ep_hbm.at[pl.ds(ebase + ch * _K, _K)],
                             ep_v.at[sl], sem_e[sl])

        def wait_in(ch, sl):
            pltpu.make_async_copy(x_hbm.at[srcv.at[sl]], gx_v.at[sl],
                                  sem_g[sl]).wait()
            pltpu.make_async_copy(ep_hbm.at[pl.ds(ebase + ch * _K, _K)],
                                  ep_v.at[sl], sem_e[sl]).wait()

        def compute(sl):
            def crow(e, c2):
                for j in range(D // 16):
                    s_ = pl.ds(j * 16, 16)
                    gx_v[sl, e, s_] = jnp.maximum(
                        gx_v[sl, e, s_] + ep_v[sl, e, s_], 0.0)
                return c2

            lax.fori_loop(0, _K, crow, 0)

        def scatter(sl):
            pltpu.async_copy(gx_v.at[sl], acc_sh.at[dstv.at[sl]],
                             sem_s[sl], add=True)

        def wait_scatter(sl):
            pltpu.make_async_copy(gx_v.at[sl], acc_sh.at[dstv.at[sl]],
                                  sem_s[sl]).wait()

        issue_src(0, 0)
        wait_src(0, 0)
        issue_in(0, 0)
        issue_src(1, 1)
        pltpu.sync_copy(dst_hbm.at[pl.ds(ebase, _K)], dstv.at[0])
        pltpu.async_copy(gx_v.at[1], acc_sh.at[dstv.at[0]], sem_s1,
                         add=True)
        issue_dst(0, 0)
        plsc.subcore_barrier()

        def phase(ph, sl, not_g, not_s):
            nsl = 1 - sl
            wait_scatter(nsl)

            @pl.when(not_g)
            def _():
                wait_src(ph + 1, nsl)
                issue_in(ph + 1, nsl)

            wait_in(ph, sl)

            @pl.when(not_s)
            def _():
                issue_src(ph + 2, sl)

            @pl.when(not_g)
            def _():
                issue_dst(ph + 1, nsl)

            compute(sl)
            wait_dst(ph, sl)
            scatter(sl)

        t_last = _NCH // 2 - 1

        def step(t, carry):
            a_not_s = t < t_last
            b_not = t < t_last
            phase(2 * t, 0, jnp.bool_(True), a_not_s)
            phase(2 * t + 1, 1, b_not, b_not)
            return carry

        lax.fori_loop(0, _NCH // 2, step, 0)
        wait_scatter(1)
        plsc.subcore_barrier()

        def dump(t, carry):
            r0 = (sid + t * _NSUB) * _K
            pltpu.sync_copy(acc_sh.at[pl.ds(r0, _K)],
                            out_hbm.at[cid, pl.ds(r0, _K)])
            return carry

        lax.fori_loop(0, nz, dump, 0)

    return k(x_all, src_all, dst_all, ep_all)


def kernel(x_author, x_paper, edge_index_a2p, edge_index_p2a,
           edge_attr_a2p, edge_attr_p2a, W_edge, b_edge, W_nn, b_nn, eps):
    src_all = jnp.concatenate([
        edge_index_a2p[0].astype(jnp.int32),
        edge_index_p2a[0].astype(jnp.int32) + N,
    ])
    dst_all = jnp.concatenate([
        edge_index_a2p[1].astype(jnp.int32),
        edge_index_p2a[1].astype(jnp.int32),
    ])
    ea_all = jnp.concatenate([edge_attr_a2p, edge_attr_p2a], axis=0)
    ep_all = _eproj(ea_all, W_edge, b_edge)

    xa, xp = x_author, x_paper
    for layer in range(2):
        x_all = jnp.concatenate([xa, xp], axis=0)
        agg = _sc_aggregate(x_all, src_all, dst_all, ep_all)
        new_stack = _out_transform(xp, xa, agg, W_nn, b_nn, eps,
                                   relu=(layer == 0))
        xp, xa = new_stack[:N], new_stack[N:]
    return (xa, xp)
